# baseline (speedup 1.0000x reference)
import dataclasses
import functools

import jax
import jax.numpy as jnp
import numpy as np
from jax import lax
from jax.experimental import pallas as pl
from jax.experimental.pallas import tpu as pltpu
from jax.experimental.pallas import tpu_sc as plsc

_NUM_BINS = 15
_NUM_CLASSES = 4
_NW = 3
_SMOOTH = 1e-8
_B = 8
_H = 512
_W = 512
_HW = _H * _W
_NPIX = _B * _HW
_NOUT = _NPIX * _NUM_CLASSES

_NUM_WORKERS = 32
_PER_W = _NPIX // _NUM_WORKERS
_CHUNK = 4096
_NCHUNK = _PER_W // _CHUNK
_TAB_PAD = 136


def _code_body(x_ref, o_ref):
    f32 = jnp.float32
    xs = [x_ref[0, c] for c in range(_NUM_CLASSES)]
    m = jnp.maximum(jnp.maximum(xs[0], xs[1]), jnp.maximum(xs[2], xs[3]))
    es = [jnp.exp(x - m) for x in xs]
    s = (es[0] + es[1]) + (es[2] + es[3])
    ps = [e / s for e in es]
    p0, p1, p2, p3 = ps
    bm = [
        (p0 >= p1) & (p0 >= p2) & (p0 >= p3),
        (p1 > p0) & (p1 >= p2) & (p1 >= p3),
        (p2 > p0) & (p2 > p1) & (p2 >= p3),
        (p3 > p0) & (p3 > p1) & (p3 > p2),
    ]
    width = np.float32(1.0 / _NUM_BINS)
    zc = jnp.zeros((_H, 1), f32)
    zr = jnp.zeros((1, _W), f32)
    packed = None
    for c in range(_NUM_CLASSES):
        bf = jnp.where(bm[c], f32(1.0), f32(0.0))
        rs = bf + jnp.concatenate([bf[:, 1:], zc], axis=1) \
                + jnp.concatenate([zc, bf[:, :-1]], axis=1)
        cs = rs + jnp.concatenate([rs[1:, :], zr], axis=0) \
                + jnp.concatenate([zr, rs[:-1, :]], axis=0)
        cnt = (cs - bf).astype(jnp.int32)
        bin_i = jnp.clip(jnp.floor(ps[c] / width).astype(jnp.int32), 0,
                         _NUM_BINS - 1)
        code = cnt * _NUM_BINS + bin_i
        shifted = code if c == 0 else lax.shift_left(code, jnp.int32(8 * c))
        packed = shifted if packed is None else (packed | shifted)
    o_ref[0] = packed


def _sc_body(packed_hbm, tab_hbm, out_hbm, in_v, t0, t1, t2, t3,
             ov0, ov1, ov2, ov3):
    f32 = jnp.float32
    wid = lax.axis_index("s") * 2 + lax.axis_index("c")
    base = wid * _PER_W
    b = wid // _NUM_CLASSES
    q = wid % _NUM_CLASSES
    pltpu.sync_copy(tab_hbm.at[0], t0)
    pltpu.sync_copy(tab_hbm.at[1], t1)
    pltpu.sync_copy(tab_hbm.at[2], t2)
    pltpu.sync_copy(tab_hbm.at[3], t3)
    tabs = (t0, t1, t2, t3)
    ovs = (ov0, ov1, ov2, ov3)
    mask8 = jnp.full((16,), 255, jnp.int32)

    @pl.loop(0, _NCHUNK)
    def _chunk(k):
        src = base + k * _CHUNK
        pltpu.sync_copy(packed_hbm.at[pl.ds(src, _CHUNK)], in_v)

        @pl.loop(0, _CHUNK // 16)
        def _vec(i):
            off = i * 16
            v = in_v[pl.ds(off, 16)]
            c0 = v & mask8
            c1 = lax.shift_right_logical(v, jnp.int32(8)) & mask8
            c2 = lax.shift_right_logical(v, jnp.int32(16)) & mask8
            c3 = lax.shift_right_logical(v, jnp.int32(24))
            g0 = plsc.load_gather(t0, [c0])
            g1 = plsc.load_gather(t1, [c1])
            g2 = plsc.load_gather(t2, [c2])
            g3 = plsc.load_gather(t3, [c3])
            sv = (g0 + g1) + (g2 + g3)
            sv = jnp.where(sv == f32(0.0), f32(_SMOOTH), sv)
            r = f32(1.0) / sv
            ov0[pl.ds(off, 16)] = g0 * r
            ov1[pl.ds(off, 16)] = g1 * r
            ov2[pl.ds(off, 16)] = g2 * r
            ov3[pl.ds(off, 16)] = g3 * r

        hw_off = q * _PER_W + k * _CHUNK
        for c in range(_NUM_CLASSES):
            dst = (b * _NUM_CLASSES + c) * _HW + hw_off
            pltpu.sync_copy(ovs[c], out_hbm.at[pl.ds(dst, _CHUNK)])


def kernel(logits, val_freqs):
    packed = pl.pallas_call(
        _code_body,
        grid=(_B,),
        in_specs=[pl.BlockSpec((1, _NUM_CLASSES, _H, _W),
                               lambda i: (i, 0, 0, 0))],
        out_specs=pl.BlockSpec((1, _H, _W), lambda i: (i, 0, 0)),
        out_shape=jax.ShapeDtypeStruct((_B, _H, _W), jnp.int32),
    )(logits)
    packed_flat = packed.reshape(_NPIX)

    tab = jnp.zeros((_NUM_CLASSES, _TAB_PAD), jnp.float32)
    tab = tab.at[:, : _NW * _NW * _NUM_BINS].set(
        val_freqs.reshape(_NUM_CLASSES, _NW * _NW * _NUM_BINS))

    mesh = plsc.VectorSubcoreMesh(core_axis_name="c", subcore_axis_name="s")
    cp = pltpu.CompilerParams()
    if "needs_layout_passes" in pltpu.CompilerParams.__dataclass_fields__:
        cp = dataclasses.replace(cp, needs_layout_passes=False)
    sc = pl.kernel(
        _sc_body,
        out_type=jax.ShapeDtypeStruct((_NOUT,), jnp.float32),
        mesh=mesh,
        scratch_types=[
            pltpu.VMEM((_CHUNK,), jnp.int32),
            pltpu.VMEM((_TAB_PAD,), jnp.float32),
            pltpu.VMEM((_TAB_PAD,), jnp.float32),
            pltpu.VMEM((_TAB_PAD,), jnp.float32),
            pltpu.VMEM((_TAB_PAD,), jnp.float32),
            pltpu.VMEM((_CHUNK,), jnp.float32),
            pltpu.VMEM((_CHUNK,), jnp.float32),
            pltpu.VMEM((_CHUNK,), jnp.float32),
            pltpu.VMEM((_CHUNK,), jnp.float32),
        ],
        compiler_params=cp,
    )
    out_flat = sc(packed_flat, tab)
    return out_flat.reshape(_B, _NUM_CLASSES, _H, _W)
